# Initial kernel scaffold; baseline (speedup 1.0000x reference)
import jax
import jax.numpy as jnp
from jax.experimental import pallas as pl


def kernel(target_probs, bonus_token_ids, draft_probs, draft_token_ids):
    raise NotImplementedError("write your pallas kernel here")


# speedup vs baseline: 1.0495x; 1.0495x over previous
import functools

import jax
import jax.numpy as jnp
from jax.experimental import pallas as pl
from jax.experimental.pallas import tpu as pltpu

import numpy as np

_TINY = np.float32(np.finfo(np.float32).tiny)


def _scan_body(V, C, tok_ref, t_ref, d_ref, q_ref,
               idx_out, selt_out, seld_out,
               run_max, run_idx, acc_t, acc_d):
    j = pl.program_id(0)
    R = t_ref.shape[0]

    @pl.when(j == 0)
    def _init():
        run_max[...] = jnp.full_like(run_max, -jnp.inf)
        run_idx[...] = jnp.zeros_like(run_idx)
        acc_t[...] = jnp.zeros_like(acc_t)
        acc_d[...] = jnp.zeros_like(acc_d)

    cols = jax.lax.broadcasted_iota(jnp.int32, (R, C), 1) + j * C
    valid = cols < V
    t = t_ref[...]
    d = d_ref[...]
    q = q_ref[...]
    f = jnp.maximum(t - d, _TINY)
    s = jnp.where(valid, f / q, jnp.float32(-1.0))

    cmax = jnp.max(s, axis=1, keepdims=True)
    carg = jnp.min(jnp.where(s == cmax, cols, jnp.int32(2**30)),
                   axis=1, keepdims=True)

    tok = tok_ref[:, 0:1]
    m = cols == tok
    st = jnp.sum(jnp.where(m, t, 0.0), axis=1, keepdims=True)
    sd = jnp.sum(jnp.where(m, d, 0.0), axis=1, keepdims=True)

    upd = cmax > run_max[:, 0:1]
    run_max[:, 0:1] = jnp.where(upd, cmax, run_max[:, 0:1])
    run_idx[:, 0:1] = jnp.where(upd, carg, run_idx[:, 0:1])
    acc_t[:, 0:1] = acc_t[:, 0:1] + st
    acc_d[:, 0:1] = acc_d[:, 0:1] + sd

    @pl.when(j == pl.num_programs(0) - 1)
    def _fin():
        idx_out[...] = run_idx[...]
        selt_out[...] = acc_t[...]
        seld_out[...] = acc_d[...]


def _out_body(B, K, u_ref, selt_ref, seld_ref, draft9_ref, rec9_ref,
              bonus_ref, out_ref):
    u = u_ref[...]
    ratio = jnp.minimum(selt_ref[...] / seld_ref[...], jnp.float32(1.0))
    rej = jnp.logical_not(u < ratio)
    kidx = jax.lax.broadcasted_iota(jnp.int32, (B, K), 1)
    limit = jnp.min(jnp.where(rej, kidx, jnp.int32(K)),
                    axis=1, keepdims=True)

    k9 = jax.lax.broadcasted_iota(jnp.int32, (B, K + 1), 1)
    draft9 = draft9_ref[...]
    rec9 = rec9_ref[...]
    bonus = jnp.broadcast_to(bonus_ref[...], (B, K + 1))
    neg1 = jnp.full((B, K + 1), -1, jnp.int32)

    inner = jnp.where(k9 < limit, draft9,
                      jnp.where(k9 == limit, rec9, neg1))
    out_ref[...] = jnp.where(k9 == K,
                             jnp.where(limit == K, bonus, neg1),
                             inner)


def kernel(target_probs, bonus_token_ids, draft_probs, draft_token_ids):
    B, K, V = target_probs.shape
    R = B * K

    rkey = jax.random.key(1)
    ku, kq = jax.random.split(rkey)
    u = jax.random.uniform(ku, (B, K), dtype=jnp.float32)
    q = jax.random.exponential(kq, (R, V), dtype=jnp.float32)

    t2 = target_probs.reshape(R, V)
    d2 = draft_probs.reshape(R, V)
    tok_b = jnp.broadcast_to(draft_token_ids.reshape(R, 1), (R, 128))

    C = 2048 if V >= 2048 else ((V + 127) // 128) * 128
    nchunks = (V + C - 1) // C

    idx, selt, seld = pl.pallas_call(
        functools.partial(_scan_body, V, C),
        grid=(nchunks,),
        in_specs=[
            pl.BlockSpec((R, 128), lambda j: (0, 0)),
            pl.BlockSpec((R, C), lambda j: (0, j)),
            pl.BlockSpec((R, C), lambda j: (0, j)),
            pl.BlockSpec((R, C), lambda j: (0, j)),
        ],
        out_specs=[
            pl.BlockSpec((R, 128), lambda j: (0, 0)),
            pl.BlockSpec((R, 128), lambda j: (0, 0)),
            pl.BlockSpec((R, 128), lambda j: (0, 0)),
        ],
        out_shape=[
            jax.ShapeDtypeStruct((R, 128), jnp.int32),
            jax.ShapeDtypeStruct((R, 128), jnp.float32),
            jax.ShapeDtypeStruct((R, 128), jnp.float32),
        ],
        scratch_shapes=[
            pltpu.VMEM((R, 128), jnp.float32),
            pltpu.VMEM((R, 128), jnp.int32),
            pltpu.VMEM((R, 128), jnp.float32),
            pltpu.VMEM((R, 128), jnp.float32),
        ],
    )(tok_b, t2, d2, q)

    rec = idx[:, 0].reshape(B, K)
    sel_t = selt[:, 0].reshape(B, K)
    sel_d = seld[:, 0].reshape(B, K)

    pad = ((0, 0), (0, 1))
    draft9 = jnp.pad(draft_token_ids, pad)
    rec9 = jnp.pad(rec, pad)

    out = pl.pallas_call(
        functools.partial(_out_body, B, K),
        out_shape=jax.ShapeDtypeStruct((B, K + 1), jnp.int32),
    )(u, sel_t, sel_d, draft9, rec9, bonus_token_ids)
    return out


# speedup vs baseline: 1.1323x; 1.0789x over previous
import functools

import jax
import jax.numpy as jnp
from jax.experimental import pallas as pl
from jax.experimental.pallas import tpu as pltpu

import numpy as np

_TINY = np.float32(np.finfo(np.float32).tiny)


def _threefry2x32(k0, k1, x0, x1):
    k2 = k0 ^ k1 ^ np.uint32(0x1BD11BDA)
    ks = (k0, k1, k2)
    rot = ((13, 15, 26, 6), (17, 29, 16, 24))
    x0 = x0 + k0
    x1 = x1 + k1
    for i in range(5):
        for d in rot[i % 2]:
            x0 = x0 + x1
            x1 = (x1 << np.uint32(d)) | (x1 >> np.uint32(32 - d))
            x1 = x1 ^ x0
        x0 = x0 + ks[(i + 1) % 3]
        x1 = x1 + ks[(i + 2) % 3] + np.uint32(i + 1)
    return x0, x1


def _exp_from_bits(bits):
    fb = (bits >> np.uint32(9)) | np.uint32(0x3F800000)
    u = jax.lax.bitcast_convert_type(fb, jnp.float32) - jnp.float32(1.0)
    return -jnp.log1p(-u)


def _scan_body(V, C, tok_ref, key_ref, t_ref, d_ref,
               idx_out, selt_out, seld_out,
               run_max, run_idx, acc_t, acc_d):
    j = pl.program_id(0)
    R = t_ref.shape[0]

    @pl.when(j == 0)
    def _init():
        run_max[...] = jnp.full_like(run_max, -jnp.inf)
        run_idx[...] = jnp.zeros_like(run_idx)
        acc_t[...] = jnp.zeros_like(acc_t)
        acc_d[...] = jnp.zeros_like(acc_d)

    cols = jax.lax.broadcasted_iota(jnp.int32, (R, C), 1) + j * C
    valid = cols < V
    t = t_ref[...]
    d = d_ref[...]

    rows = jax.lax.broadcasted_iota(jnp.int32, (R, C), 0)
    x1 = (rows * V + cols).astype(jnp.uint32)
    o0, o1 = _threefry2x32(key_ref[0], key_ref[1], np.uint32(0), x1)
    q = _exp_from_bits(o0 ^ o1)

    f = jnp.maximum(t - d, _TINY)
    s = jnp.where(valid, f / q, jnp.float32(-1.0))

    cmax = jnp.max(s, axis=1, keepdims=True)
    carg = jnp.min(jnp.where(s == cmax, cols, jnp.int32(2**30)),
                   axis=1, keepdims=True)

    tok = tok_ref[:, 0:1]
    m = cols == tok
    st = jnp.sum(jnp.where(m, t, 0.0), axis=1, keepdims=True)
    sd = jnp.sum(jnp.where(m, d, 0.0), axis=1, keepdims=True)

    upd = cmax > run_max[:, 0:1]
    run_max[:, 0:1] = jnp.where(upd, cmax, run_max[:, 0:1])
    run_idx[:, 0:1] = jnp.where(upd, carg, run_idx[:, 0:1])
    acc_t[:, 0:1] = acc_t[:, 0:1] + st
    acc_d[:, 0:1] = acc_d[:, 0:1] + sd

    @pl.when(j == pl.num_programs(0) - 1)
    def _fin():
        idx_out[...] = run_idx[...]
        selt_out[...] = acc_t[...]
        seld_out[...] = acc_d[...]


def _out_body(B, K, u_ref, selt_ref, seld_ref, draft9_ref, rec9_ref,
              bonus_ref, out_ref):
    u = u_ref[...]
    ratio = jnp.minimum(selt_ref[...] / seld_ref[...], jnp.float32(1.0))
    rej = jnp.logical_not(u < ratio)
    kidx = jax.lax.broadcasted_iota(jnp.int32, (B, K), 1)
    limit = jnp.min(jnp.where(rej, kidx, jnp.int32(K)),
                    axis=1, keepdims=True)

    k9 = jax.lax.broadcasted_iota(jnp.int32, (B, K + 1), 1)
    draft9 = draft9_ref[...]
    rec9 = rec9_ref[...]
    bonus = jnp.broadcast_to(bonus_ref[...], (B, K + 1))
    neg1 = jnp.full((B, K + 1), -1, jnp.int32)

    inner = jnp.where(k9 < limit, draft9,
                      jnp.where(k9 == limit, rec9, neg1))
    out_ref[...] = jnp.where(k9 == K,
                             jnp.where(limit == K, bonus, neg1),
                             inner)


def kernel(target_probs, bonus_token_ids, draft_probs, draft_token_ids):
    B, K, V = target_probs.shape
    R = B * K

    rkey = jax.random.key(1)
    ku, kq = jax.random.split(rkey)
    u = jax.random.uniform(ku, (B, K), dtype=jnp.float32)
    kq_data = jax.random.key_data(kq)

    t2 = target_probs.reshape(R, V)
    d2 = draft_probs.reshape(R, V)
    tok_b = jnp.broadcast_to(draft_token_ids.reshape(R, 1), (R, 128))

    C = 2048 if V >= 2048 else ((V + 127) // 128) * 128
    nchunks = (V + C - 1) // C

    idx, selt, seld = pl.pallas_call(
        functools.partial(_scan_body, V, C),
        grid=(nchunks,),
        in_specs=[
            pl.BlockSpec((R, 128), lambda j: (0, 0)),
            pl.BlockSpec(memory_space=pltpu.SMEM),
            pl.BlockSpec((R, C), lambda j: (0, j)),
            pl.BlockSpec((R, C), lambda j: (0, j)),
        ],
        out_specs=[
            pl.BlockSpec((R, 128), lambda j: (0, 0)),
            pl.BlockSpec((R, 128), lambda j: (0, 0)),
            pl.BlockSpec((R, 128), lambda j: (0, 0)),
        ],
        out_shape=[
            jax.ShapeDtypeStruct((R, 128), jnp.int32),
            jax.ShapeDtypeStruct((R, 128), jnp.float32),
            jax.ShapeDtypeStruct((R, 128), jnp.float32),
        ],
        scratch_shapes=[
            pltpu.VMEM((R, 128), jnp.float32),
            pltpu.VMEM((R, 128), jnp.int32),
            pltpu.VMEM((R, 128), jnp.float32),
            pltpu.VMEM((R, 128), jnp.float32),
        ],
    )(tok_b, kq_data, t2, d2)

    rec = idx[:, 0].reshape(B, K)
    sel_t = selt[:, 0].reshape(B, K)
    sel_d = seld[:, 0].reshape(B, K)

    pad = ((0, 0), (0, 1))
    draft9 = jnp.pad(draft_token_ids, pad)
    rec9 = jnp.pad(rec, pad)

    out = pl.pallas_call(
        functools.partial(_out_body, B, K),
        out_shape=jax.ShapeDtypeStruct((B, K + 1), jnp.int32),
    )(u, sel_t, sel_d, draft9, rec9, bonus_token_ids)
    return out
